# baseline (speedup 1.0000x reference)
import functools

import jax
import jax.numpy as jnp
from jax import lax
from jax.experimental import pallas as pl
from jax.experimental.pallas import tpu as pltpu
from jax.experimental.pallas import tpu_sc as plsc

_N = 100000
_NW = 32
_ROWS_PER_W = _N // _NW
_R = 125
_STEPS = _ROWS_PER_W // _R

_SLICES = [
    (0, 0, 64, 0), (128, 64, 96, 0), (320, 160, 80, 0),
    (64, 0, 64, 1), (224, 64, 96, 1), (400, 160, 80, 1),
]


def _sc_body(x, o0, o1, b0, b1, sem):
    wid = lax.axis_index("s") * 2 + lax.axis_index("c")
    base = wid * _ROWS_PER_W
    bufs = (b0, b1)
    outs = (o0, o1)

    def step(i, carry):
        r0 = base + i * _R
        copies = [
            pltpu.make_async_copy(
                x.at[pl.ds(r0, _R), pl.ds(s, w)],
                bufs[oi].at[:, pl.ds(d, w)], sem)
            for (s, d, w, oi) in _SLICES
        ]
        for c in copies:
            c.start()
        for c in copies:
            c.wait()
        out_copies = [
            pltpu.make_async_copy(bufs[k], outs[k].at[pl.ds(r0, _R)], sem)
            for k in range(2)
        ]
        for c in out_copies:
            c.start()
        for c in out_copies:
            c.wait()
        return carry

    lax.fori_loop(0, _STEPS, step, 0)


def kernel(x):
    n, _ = x.shape
    run = functools.partial(
        pl.kernel,
        out_type=[jax.ShapeDtypeStruct((n, 240), jnp.float32)] * 2,
        mesh=plsc.VectorSubcoreMesh(core_axis_name="c", subcore_axis_name="s"),
        scratch_types=[
            pltpu.VMEM((_R, 240), jnp.float32),
            pltpu.VMEM((_R, 240), jnp.float32),
            pltpu.SemaphoreType.DMA,
        ],
        compiler_params=pltpu.CompilerParams(use_tc_tiling_on_sc=False),
    )(_sc_body)
    o0, o1 = run(x)
    return (o0, o1)


# speedup vs baseline: 1.0136x; 1.0136x over previous
import jax
import jax.numpy as jnp
from jax import lax
from jax.experimental import pallas as pl
from jax.experimental.pallas import tpu as pltpu
from jax.experimental.pallas import tpu_sc as plsc

_N = 100000
_NW = 32
_ROWS_PER_W = _N // _NW
_R = 125
_STEPS = _ROWS_PER_W // _R
_L = 16

_PERM = (
    [0, 1, 2, 3] + [15, 16, 17, 18]
    + [4, 5, 6, 7, 8, 9] + [19, 20, 21, 22, 23, 24]
    + [10, 11, 12, 13, 14] + [25, 26, 27, 28, 29]
)


def _permute_block(buf):
    def row(r, carry):
        vals = [buf[r, pl.ds(_L * u, _L)] for u in range(30)]
        for u, v in zip(_PERM, vals):
            buf[r, pl.ds(_L * u, _L)] = v
        return carry
    lax.fori_loop(0, _R, row, 0)


def _sc_body(x, o0, o1, buf_a, buf_b, sem_in, sem_out):
    wid = lax.axis_index("s") * 2 + lax.axis_index("c")
    base = wid * _ROWS_PER_W
    bufs = (buf_a, buf_b)

    def in_copy(step, b):
        r0 = base + step * _R
        return pltpu.make_async_copy(x.at[pl.ds(r0, _R)], bufs[b], sem_in.at[b])

    def out_copies(step, b):
        r0 = base + step * _R
        return (
            pltpu.make_async_copy(
                bufs[b].at[:, pl.ds(0, 240)], o0.at[pl.ds(r0, _R)], sem_out.at[b]),
            pltpu.make_async_copy(
                bufs[b].at[:, pl.ds(240, 240)], o1.at[pl.ds(r0, _R)], sem_out.at[b]),
        )

    pending_in = {0: in_copy(0, 0)}
    pending_in[0].start()
    prev_out = None
    for i in range(_STEPS):
        b = i % 2
        if i + 1 < _STEPS:
            if prev_out is not None:
                for c in prev_out:
                    c.wait()
                prev_out = None
            nxt = in_copy(i + 1, 1 - b)
            nxt.start()
            pending_in[i + 1] = nxt
        pending_in.pop(i).wait()
        _permute_block(bufs[b])
        cs = out_copies(i, b)
        for c in cs:
            c.start()
        if prev_out is not None:
            for c in prev_out:
                c.wait()
        prev_out = cs
    for c in prev_out:
        c.wait()


def kernel(x):
    n, _ = x.shape
    run = pl.kernel(
        _sc_body,
        out_type=[jax.ShapeDtypeStruct((n, 240), jnp.float32)] * 2,
        mesh=plsc.VectorSubcoreMesh(core_axis_name="c", subcore_axis_name="s"),
        scratch_types=[
            pltpu.VMEM((_R, 480), jnp.float32),
            pltpu.VMEM((_R, 480), jnp.float32),
            pltpu.SemaphoreType.DMA((2,)),
            pltpu.SemaphoreType.DMA((2,)),
        ],
        compiler_params=pltpu.CompilerParams(use_tc_tiling_on_sc=False),
    )
    o0, o1 = run(x)
    return (o0, o1)


# speedup vs baseline: 3.3982x; 3.3528x over previous
import jax
import jax.numpy as jnp
from jax import lax
from jax.experimental import pallas as pl
from jax.experimental.pallas import tpu as pltpu
from jax.experimental.pallas import tpu_sc as plsc

_N = 100000
_NW = 32
_R = 40
_NBLK = _N // _R
_PAIRS = (_NBLK // _NW + 2) // 2
_L = 16

_UNIT_MAP = (
    [(0, u) for u in range(4)] + [(1, u) for u in range(4)]
    + [(0, 4 + u) for u in range(6)] + [(1, 4 + u) for u in range(6)]
    + [(0, 10 + u) for u in range(5)] + [(1, 10 + u) for u in range(5)]
)


def _permute_block(xb, b0, b1):
    dsts = (b0, b1)

    def row(r, carry):
        vals = [xb[r, pl.ds(_L * u, _L)] for u in range(30)]
        for (oi, d), v in zip(_UNIT_MAP, vals):
            dsts[oi][r, pl.ds(_L * d, _L)] = v
        return carry

    lax.fori_loop(0, _R, row, 0)


def _sc_body(x, o0, o1, xb_a, b0_a, b1_a, xb_b, b0_b, b1_b, sem_in, sem_out):
    wid = lax.axis_index("s") * 2 + lax.axis_index("c")
    sets = ((xb_a, b0_a, b1_a), (xb_b, b0_b, b1_b))

    def blk(step):
        return wid + _NW * step

    def valid(step):
        return jnp.logical_and(step >= 0, blk(step) < _NBLK)

    def in_copy(step, s):
        r0 = blk(step) * _R
        return pltpu.make_async_copy(x.at[pl.ds(r0, _R)], sets[s][0],
                                     sem_in.at[s])

    def out_copies(step, s):
        r0 = blk(step) * _R
        return (
            pltpu.make_async_copy(sets[s][1], o0.at[pl.ds(r0, _R)],
                                  sem_out.at[s]),
            pltpu.make_async_copy(sets[s][2], o1.at[pl.ds(r0, _R)],
                                  sem_out.at[s]),
        )

    def when(cond, fn):
        def wrapped():
            fn()
        pl.when(cond)(wrapped)

    in_copy(0, 0).start()

    def body(k, carry):
        s_a = 2 * k
        s_b = 2 * k + 1
        when(valid(s_a), lambda: in_copy(s_a, 0).wait())
        when(valid(s_a), lambda: _permute_block(*sets[0]))
        when(valid(s_b - 2), lambda: [c.wait() for c in out_copies(s_b - 2, 1)])
        when(valid(s_b), lambda: in_copy(s_b, 1).start())
        when(valid(s_a), lambda: [c.start() for c in out_copies(s_a, 0)])
        when(valid(s_b), lambda: in_copy(s_b, 1).wait())
        when(valid(s_b), lambda: _permute_block(*sets[1]))
        when(valid(s_a), lambda: [c.wait() for c in out_copies(s_a, 0)])
        when(valid(s_a + 2), lambda: in_copy(s_a + 2, 0).start())
        when(valid(s_b), lambda: [c.start() for c in out_copies(s_b, 1)])
        return carry

    lax.fori_loop(0, _PAIRS, body, 0)


def kernel(x):
    n, _ = x.shape
    run = pl.kernel(
        _sc_body,
        out_type=[jax.ShapeDtypeStruct((n, 240), jnp.float32)] * 2,
        mesh=plsc.VectorSubcoreMesh(core_axis_name="c", subcore_axis_name="s"),
        scratch_types=[
            pltpu.VMEM((_R, 480), jnp.float32),
            pltpu.VMEM((_R, 240), jnp.float32),
            pltpu.VMEM((_R, 240), jnp.float32),
            pltpu.VMEM((_R, 480), jnp.float32),
            pltpu.VMEM((_R, 240), jnp.float32),
            pltpu.VMEM((_R, 240), jnp.float32),
            pltpu.SemaphoreType.DMA((2,)),
            pltpu.SemaphoreType.DMA((2,)),
        ],
        compiler_params=pltpu.CompilerParams(use_tc_tiling_on_sc=True),
    )
    o0, o1 = run(x)
    return (o0, o1)
